# baseline (speedup 1.0000x reference)
import numpy as np
import jax
import jax.numpy as jnp
from jax.experimental import pallas as pl
from jax.experimental.pallas import tpu as pltpu

_N = 128
_D = 1024
_ED = 256
_L = 3
_E = _N * (_N - 1)
_TL = 32
_NT = _N // _TL
_NET = 2 * _NT
_TE = _E // _NET
_C = 8


def _gelu(x):
    return x * (0.5 * jax.lax.erf(x * np.float32(1.0 / np.sqrt(2.0))) + 0.5)


def _bf(x):
    return x.astype(jnp.bfloat16)


def _dot(a, b):
    return jnp.dot(a, b, preferred_element_type=jnp.float32)


def _main_body(iidx_ref, jidx_ref, tracks_ref, W_ne_ref, b_ne_ref,
               W_ep1_ref, b_ep1_ref, W_ep2_ref, b_ep2_ref, W_ee_ref,
               b_ee_ref, W_em_ref, b_em_ref,
               wnm_ref, b_nm_ref, ln_g_ref, ln_b_ref,
               W_gp1_ref, b_gp1_ref, W_gp2_ref, b_gp2_ref,
               eout_ref, x_out_ref, ge_out_ref,
               x_ref, acc_ref, a_s, b_s, abf_s, bbf_s, bee_s, efd_s,
               wc_s, bc_s, gd_s):
    l = pl.program_id(0)
    t = pl.program_id(1)

    @pl.when((l == 0) & (t == 0))
    def _():
        nodes = _dot(tracks_ref[...], W_ne_ref[...]) + b_ne_ref[...]
        x_ref[...] = nodes
        a = _dot(nodes, W_ep1_ref[:_D, :]) + b_ep1_ref[...]
        b = _dot(nodes, W_ep1_ref[_D:, :])
        a_s[...] = a
        b_s[...] = b
        abf_s[...] = _bf(a)
        bbf_s[...] = _bf(b)
        bee_s[...] = _dot(b_ep2_ref[...], W_ee_ref[...]) + b_ee_ref[...]
        efd_s[...] = _bf(_dot(_bf(_gelu(a + b)), _bf(W_ep2_ref[...])))

    @pl.when(t == 0)
    def _():
        wc = _dot(_bf(W_ee_ref[...]), W_em_ref[0])
        bc = _dot(_bf(bee_s[...]), W_em_ref[0]) + b_em_ref[0]
        wcb = _bf(wc)
        wc_s[...] = wcb
        bc_s[...] = bc
        gd_s[...] = _gelu(_dot(efd_s[...], wcb) + bc)

    @pl.when(l < 2)
    def _():
        col = jax.lax.broadcasted_iota(jnp.int32, (_TE, _N), 1)
        pi = (col == iidx_ref[0, 0, :][:, None]).astype(jnp.bfloat16)
        pj = (col == jidx_ref[0, 0, :][:, None]).astype(jnp.bfloat16)
        pre = _dot(pi, abf_s[...]) + _dot(pj, bbf_s[...])
        ef0 = _dot(_bf(_gelu(pre)), _bf(W_ep2_ref[...]))
        eout_ref[...] = _dot(_bf(ef0), _bf(W_ee_ref[...])) + bee_s[...]

    b = b_s[...]
    xt = x_ref[pl.ds(t * _TL, _TL), :]
    wp2b = _bf(W_ep2_ref[...])
    contrib = None
    for c in range(_TL // _C):
        ac = a_s[pl.ds(t * _TL + c * _C, _C), :]
        pre = ac[:, None, :] + b[None, :, :]
        ef0 = _dot(_bf(_gelu(pre.reshape(_C * _N, _ED))), wp2b)
        g = _gelu(_dot(_bf(ef0), wc_s[...]) + bc_s[...])
        g = g.reshape(_C, _N, _D)
        xc = xt[c * _C:(c + 1) * _C, :]
        part = jnp.sum(g * xc[:, None, :], axis=0)
        contrib = part if contrib is None else contrib + part

    @pl.when(t == 0)
    def _():
        acc_ref[...] = contrib - x_ref[...] * gd_s[...]

    @pl.when(t > 0)
    def _():
        acc_ref[...] = acc_ref[...] + contrib

    @pl.when(t == _NT - 1)
    def _():
        agg = acc_ref[...] * (1.0 / 127.0)
        x = x_ref[...]
        wnm = wnm_ref[0]
        h = (_dot(_bf(x), wnm[:_D, :])
             + _dot(_bf(agg), wnm[_D:, :])
             + b_nm_ref[0])
        mu = jnp.mean(h, axis=1, keepdims=True)
        var = jnp.mean((h - mu) * (h - mu), axis=1, keepdims=True)
        hn = (h - mu) / jnp.sqrt(var + 1e-5) * ln_g_ref[0] + ln_b_ref[0]
        xn = _gelu(hn)
        x_ref[...] = xn

        @pl.when(l == _L - 1)
        def _():
            x_out_ref[...] = xn
            gmean = jnp.mean(xn, axis=0, keepdims=True)
            hp = _gelu(_dot(gmean, W_gp1_ref[...]) + b_gp1_ref[...])
            ge_out_ref[...] = _dot(hp, W_gp2_ref[...]) + b_gp2_ref[...]


_ii = np.repeat(np.arange(_N), _N)
_jj = np.tile(np.arange(_N), _N)
_msk = _ii != _jj
_SRC = np.ascontiguousarray(_ii[_msk]).astype(np.int32)
_TGT = np.ascontiguousarray(_jj[_msk]).astype(np.int32)
_IIDX = _SRC.reshape(_NET, 1, _TE)
_JIDX = _TGT.reshape(_NET, 1, _TE)


def _etile(l, t):
    return jnp.minimum(l * _NT + t, _NET - 1)


def kernel(tracks, W_ne, b_ne, W_ep1, b_ep1, W_ep2, b_ep2, W_ee, b_ee,
           W_nm, b_nm, ln_g, ln_b, W_em, b_em, W_gp1, b_gp1, W_gp2, b_gp2):
    f32 = jnp.float32
    bf16 = jnp.bfloat16

    edge_features, x, graph_embedding = pl.pallas_call(
        _main_body,
        grid=(_L, _NT),
        in_specs=[
            pl.BlockSpec((1, 1, _TE), lambda l, t: (_etile(l, t), 0, 0)),
            pl.BlockSpec((1, 1, _TE), lambda l, t: (_etile(l, t), 0, 0)),
            pl.BlockSpec((_N, _D), lambda l, t: (0, 0)),
            pl.BlockSpec((_D, _D), lambda l, t: (0, 0)),
            pl.BlockSpec((1, _D), lambda l, t: (0, 0)),
            pl.BlockSpec((2 * _D, _ED), lambda l, t: (0, 0)),
            pl.BlockSpec((1, _ED), lambda l, t: (0, 0)),
            pl.BlockSpec((_ED, _ED), lambda l, t: (0, 0)),
            pl.BlockSpec((1, _ED), lambda l, t: (0, 0)),
            pl.BlockSpec((_ED, _D), lambda l, t: (0, 0)),
            pl.BlockSpec((1, _D), lambda l, t: (0, 0)),
            pl.BlockSpec((1, _D, _D), lambda l, t: (l, 0, 0)),
            pl.BlockSpec((1, 1, _D), lambda l, t: (l, 0, 0)),
            pl.BlockSpec((1, 2 * _D, _D), lambda l, t: (l, 0, 0)),
            pl.BlockSpec((1, 1, _D), lambda l, t: (l, 0, 0)),
            pl.BlockSpec((1, 1, _D), lambda l, t: (l, 0, 0)),
            pl.BlockSpec((1, 1, _D), lambda l, t: (l, 0, 0)),
            pl.BlockSpec((_D, _D // 2), lambda l, t: (0, 0)),
            pl.BlockSpec((1, _D // 2), lambda l, t: (0, 0)),
            pl.BlockSpec((_D // 2, _D), lambda l, t: (0, 0)),
            pl.BlockSpec((1, _D), lambda l, t: (0, 0)),
        ],
        out_specs=(
            pl.BlockSpec((_TE, _D), lambda l, t: (_etile(l, t), 0)),
            pl.BlockSpec((_N, _D), lambda l, t: (0, 0)),
            pl.BlockSpec((1, _D), lambda l, t: (0, 0)),
        ),
        out_shape=(
            jax.ShapeDtypeStruct((_E, _D), f32),
            jax.ShapeDtypeStruct((_N, _D), f32),
            jax.ShapeDtypeStruct((1, _D), f32),
        ),
        scratch_shapes=[
            pltpu.VMEM((_N, _D), f32),
            pltpu.VMEM((_N, _D), f32),
            pltpu.VMEM((_N, _ED), f32),
            pltpu.VMEM((_N, _ED), f32),
            pltpu.VMEM((_N, _ED), bf16),
            pltpu.VMEM((_N, _ED), bf16),
            pltpu.VMEM((1, _D), f32),
            pltpu.VMEM((_N, _ED), bf16),
            pltpu.VMEM((_ED, _D), bf16),
            pltpu.VMEM((1, _D), f32),
            pltpu.VMEM((_N, _D), f32),
        ],
    )(jnp.asarray(_IIDX), jnp.asarray(_JIDX), tracks, W_ne,
      b_ne.reshape(1, _D), W_ep1, b_ep1.reshape(1, _ED), W_ep2,
      b_ep2.reshape(1, _ED), W_ee, b_ee.reshape(1, _D), W_em.astype(bf16),
      b_em.reshape(_L, 1, _D), W_nm.astype(bf16), b_nm.reshape(_L, 1, _D),
      ln_g.reshape(_L, 1, _D), ln_b.reshape(_L, 1, _D), W_gp1,
      b_gp1.reshape(1, _D // 2), W_gp2, b_gp2.reshape(1, _D))

    edge_index = jnp.stack([jnp.asarray(_SRC), jnp.asarray(_TGT)])

    return x, edge_features, edge_index, graph_embedding.reshape(_D)


# speedup vs baseline: 1.1177x; 1.1177x over previous
import numpy as np
import jax
import jax.numpy as jnp
from jax.experimental import pallas as pl
from jax.experimental.pallas import tpu as pltpu

_N = 128
_D = 1024
_ED = 256
_L = 3
_E = _N * (_N - 1)
_TL = 32
_NT = _N // _TL
_NET = 2 * _NT
_TE = _E // _NET
_C = 8


def _gelu(x):
    return x * (0.5 * jax.lax.erf(x * np.float32(1.0 / np.sqrt(2.0))) + 0.5)


def _bf(x):
    return x.astype(jnp.bfloat16)


def _dot(a, b):
    return jnp.dot(a, b, preferred_element_type=jnp.float32)


def _prep_body(tracks_ref, W_ne_ref, b_ne_ref, W_ep1_ref, b_ep1_ref,
               W_ep2_ref, b_ep2_ref, W_ee_ref, b_ee_ref, W_em_ref, b_em_ref,
               nodes_ref, a_ref, b_ref, wc_ref, bc_ref, gd_ref, bee_ref,
               abf_ref, bbf_ref, wp2b_ref, weeb_ref, efd_ref, bee_s):
    l = pl.program_id(0)

    @pl.when(l == 0)
    def _():
        nodes = _dot(tracks_ref[...], W_ne_ref[...]) + b_ne_ref[...]
        nodes_ref[...] = nodes
        a = _dot(nodes, W_ep1_ref[:_D, :]) + b_ep1_ref[...]
        b = _dot(nodes, W_ep1_ref[_D:, :])
        a_ref[...] = a
        b_ref[...] = b
        abf_ref[...] = _bf(a)
        bbf_ref[...] = _bf(b)
        wp2b_ref[...] = _bf(W_ep2_ref[...])
        weeb_ref[...] = _bf(W_ee_ref[...])
        bee = _dot(b_ep2_ref[...], W_ee_ref[...]) + b_ee_ref[...]
        bee_ref[...] = bee
        bee_s[...] = bee
        efd_ref[...] = _bf(_dot(_bf(_gelu(a + b)), _bf(W_ep2_ref[...])))

    wc = _dot(_bf(W_ee_ref[...]), _bf(W_em_ref[0]))
    bc = _dot(bee_s[...], W_em_ref[0]) + b_em_ref[0]
    wc_ref[0] = _bf(wc)
    bc_ref[0] = bc
    gd_ref[0] = _gelu(_dot(efd_ref[...], _bf(wc)) + bc)


def _main_body(iidx_ref, jidx_ref, a_ref, b_ref, abf_ref, bbf_ref,
               wp2b_ref, weeb_ref, bee_ref, nodes_ref,
               wc_ref, bc_ref, gd_ref, wnm_ref, b_nm_ref, ln_g_ref,
               ln_b_ref, W_gp1_ref, b_gp1_ref, W_gp2_ref, b_gp2_ref,
               eout_ref, x_out_ref, ge_out_ref, x_ref, acc_ref):
    l = pl.program_id(0)
    t = pl.program_id(1)

    @pl.when((l == 0) & (t == 0))
    def _():
        x_ref[...] = nodes_ref[...]

    @pl.when(l < 2)
    def _():
        col = jax.lax.broadcasted_iota(jnp.int32, (_TE, _N), 1)
        pi = (col == iidx_ref[0, 0, :][:, None]).astype(jnp.bfloat16)
        pj = (col == jidx_ref[0, 0, :][:, None]).astype(jnp.bfloat16)
        pre = _dot(pi, abf_ref[...]) + _dot(pj, bbf_ref[...])
        ef0 = _dot(_bf(_gelu(pre)), wp2b_ref[...])
        eout_ref[...] = _dot(_bf(ef0), weeb_ref[...]) + bee_ref[...]

    a = a_ref[...]
    b = b_ref[...]
    xt = x_ref[pl.ds(t * _TL, _TL), :]
    contrib = None
    for c in range(_TL // _C):
        ac = a[c * _C:(c + 1) * _C, :]
        pre = ac[:, None, :] + b[None, :, :]
        ef0 = _dot(_bf(_gelu(pre.reshape(_C * _N, _ED))), wp2b_ref[...])
        g = _gelu(_dot(_bf(ef0), wc_ref[0]) + bc_ref[0])
        g = g.reshape(_C, _N, _D)
        xc = xt[c * _C:(c + 1) * _C, :]
        part = jnp.sum(g * xc[:, None, :], axis=0)
        contrib = part if contrib is None else contrib + part

    @pl.when(t == 0)
    def _():
        acc_ref[...] = contrib - x_ref[...] * gd_ref[0]

    @pl.when(t > 0)
    def _():
        acc_ref[...] = acc_ref[...] + contrib

    @pl.when(t == _NT - 1)
    def _():
        agg = acc_ref[...] * (1.0 / 127.0)
        x = x_ref[...]
        wnm = wnm_ref[0]
        h = (_dot(_bf(x), _bf(wnm[:_D, :]))
             + _dot(_bf(agg), _bf(wnm[_D:, :]))
             + b_nm_ref[0])
        mu = jnp.mean(h, axis=1, keepdims=True)
        var = jnp.mean((h - mu) * (h - mu), axis=1, keepdims=True)
        hn = (h - mu) / jnp.sqrt(var + 1e-5) * ln_g_ref[0] + ln_b_ref[0]
        xn = _gelu(hn)
        x_ref[...] = xn

        @pl.when(l == _L - 1)
        def _():
            x_out_ref[...] = xn
            gmean = jnp.mean(xn, axis=0, keepdims=True)
            hp = _gelu(_dot(gmean, W_gp1_ref[...]) + b_gp1_ref[...])
            ge_out_ref[...] = _dot(hp, W_gp2_ref[...]) + b_gp2_ref[...]


_ii = np.repeat(np.arange(_N), _N)
_jj = np.tile(np.arange(_N), _N)
_msk = _ii != _jj
_SRC = np.ascontiguousarray(_ii[_msk]).astype(np.int32)
_TGT = np.ascontiguousarray(_jj[_msk]).astype(np.int32)
_IIDX = _SRC.reshape(_NET, 1, _TE)
_JIDX = _TGT.reshape(_NET, 1, _TE)


def _etile(l, t):
    return jnp.minimum(l * _NT + t, _NET - 1)


def kernel(tracks, W_ne, b_ne, W_ep1, b_ep1, W_ep2, b_ep2, W_ee, b_ee,
           W_nm, b_nm, ln_g, ln_b, W_em, b_em, W_gp1, b_gp1, W_gp2, b_gp2):
    f32 = jnp.float32
    bf16 = jnp.bfloat16

    (nodes, A, B, Wc, bc, gd, bee, A_bf, B_bf, W_ep2_bf,
     W_ee_bf) = pl.pallas_call(
        _prep_body,
        grid=(_L,),
        in_specs=[
            pl.BlockSpec((_N, _D), lambda l: (0, 0)),
            pl.BlockSpec((_D, _D), lambda l: (0, 0)),
            pl.BlockSpec((1, _D), lambda l: (0, 0)),
            pl.BlockSpec((2 * _D, _ED), lambda l: (0, 0)),
            pl.BlockSpec((1, _ED), lambda l: (0, 0)),
            pl.BlockSpec((_ED, _ED), lambda l: (0, 0)),
            pl.BlockSpec((1, _ED), lambda l: (0, 0)),
            pl.BlockSpec((_ED, _D), lambda l: (0, 0)),
            pl.BlockSpec((1, _D), lambda l: (0, 0)),
            pl.BlockSpec((1, _D, _D), lambda l: (l, 0, 0)),
            pl.BlockSpec((1, 1, _D), lambda l: (l, 0, 0)),
        ],
        out_specs=(
            pl.BlockSpec((_N, _D), lambda l: (0, 0)),
            pl.BlockSpec((_N, _ED), lambda l: (0, 0)),
            pl.BlockSpec((_N, _ED), lambda l: (0, 0)),
            pl.BlockSpec((1, _ED, _D), lambda l: (l, 0, 0)),
            pl.BlockSpec((1, 1, _D), lambda l: (l, 0, 0)),
            pl.BlockSpec((1, _N, _D), lambda l: (l, 0, 0)),
            pl.BlockSpec((1, _D), lambda l: (0, 0)),
            pl.BlockSpec((_N, _ED), lambda l: (0, 0)),
            pl.BlockSpec((_N, _ED), lambda l: (0, 0)),
            pl.BlockSpec((_ED, _ED), lambda l: (0, 0)),
            pl.BlockSpec((_ED, _D), lambda l: (0, 0)),
        ),
        out_shape=(
            jax.ShapeDtypeStruct((_N, _D), f32),
            jax.ShapeDtypeStruct((_N, _ED), f32),
            jax.ShapeDtypeStruct((_N, _ED), f32),
            jax.ShapeDtypeStruct((_L, _ED, _D), bf16),
            jax.ShapeDtypeStruct((_L, 1, _D), f32),
            jax.ShapeDtypeStruct((_L, _N, _D), f32),
            jax.ShapeDtypeStruct((1, _D), f32),
            jax.ShapeDtypeStruct((_N, _ED), bf16),
            jax.ShapeDtypeStruct((_N, _ED), bf16),
            jax.ShapeDtypeStruct((_ED, _ED), bf16),
            jax.ShapeDtypeStruct((_ED, _D), bf16),
        ),
        scratch_shapes=[pltpu.VMEM((_N, _ED), bf16), pltpu.VMEM((1, _D), f32)],
    )(tracks, W_ne, b_ne.reshape(1, _D), W_ep1, b_ep1.reshape(1, _ED),
      W_ep2, b_ep2.reshape(1, _ED), W_ee, b_ee.reshape(1, _D), W_em,
      b_em.reshape(_L, 1, _D))

    edge_features, x, graph_embedding = pl.pallas_call(
        _main_body,
        grid=(_L, _NT),
        in_specs=[
            pl.BlockSpec((1, 1, _TE), lambda l, t: (_etile(l, t), 0, 0)),
            pl.BlockSpec((1, 1, _TE), lambda l, t: (_etile(l, t), 0, 0)),
            pl.BlockSpec((_TL, _ED), lambda l, t: (t, 0)),
            pl.BlockSpec((_N, _ED), lambda l, t: (0, 0)),
            pl.BlockSpec((_N, _ED), lambda l, t: (0, 0)),
            pl.BlockSpec((_N, _ED), lambda l, t: (0, 0)),
            pl.BlockSpec((_ED, _ED), lambda l, t: (0, 0)),
            pl.BlockSpec((_ED, _D), lambda l, t: (0, 0)),
            pl.BlockSpec((1, _D), lambda l, t: (0, 0)),
            pl.BlockSpec((_N, _D), lambda l, t: (0, 0)),
            pl.BlockSpec((1, _ED, _D), lambda l, t: (l, 0, 0)),
            pl.BlockSpec((1, 1, _D), lambda l, t: (l, 0, 0)),
            pl.BlockSpec((1, _N, _D), lambda l, t: (l, 0, 0)),
            pl.BlockSpec((1, 2 * _D, _D), lambda l, t: (l, 0, 0)),
            pl.BlockSpec((1, 1, _D), lambda l, t: (l, 0, 0)),
            pl.BlockSpec((1, 1, _D), lambda l, t: (l, 0, 0)),
            pl.BlockSpec((1, 1, _D), lambda l, t: (l, 0, 0)),
            pl.BlockSpec((_D, _D // 2), lambda l, t: (0, 0)),
            pl.BlockSpec((1, _D // 2), lambda l, t: (0, 0)),
            pl.BlockSpec((_D // 2, _D), lambda l, t: (0, 0)),
            pl.BlockSpec((1, _D), lambda l, t: (0, 0)),
        ],
        out_specs=(
            pl.BlockSpec((_TE, _D), lambda l, t: (_etile(l, t), 0)),
            pl.BlockSpec((_N, _D), lambda l, t: (0, 0)),
            pl.BlockSpec((1, _D), lambda l, t: (0, 0)),
        ),
        out_shape=(
            jax.ShapeDtypeStruct((_E, _D), f32),
            jax.ShapeDtypeStruct((_N, _D), f32),
            jax.ShapeDtypeStruct((1, _D), f32),
        ),
        scratch_shapes=[pltpu.VMEM((_N, _D), f32), pltpu.VMEM((_N, _D), f32)],
    )(jnp.asarray(_IIDX), jnp.asarray(_JIDX), A, B, A_bf, B_bf,
      W_ep2_bf, W_ee_bf, bee, nodes, Wc, bc, gd, W_nm,
      b_nm.reshape(_L, 1, _D), ln_g.reshape(_L, 1, _D),
      ln_b.reshape(_L, 1, _D), W_gp1, b_gp1.reshape(1, _D // 2),
      W_gp2, b_gp2.reshape(1, _D))

    edge_index = jnp.stack([jnp.asarray(_SRC), jnp.asarray(_TGT)])

    return x, edge_features, edge_index, graph_embedding.reshape(_D)
